# baseline (speedup 1.0000x reference)
import jax
import jax.numpy as jnp
from jax import lax
from jax.experimental import pallas as pl
from jax.experimental.pallas import tpu as pltpu
from jax.experimental.pallas import tpu_sc as plsc

_S = 896
_ROWS = 56
_C = 384
_NW = 32
_PER_W = _S // _NW
_CHUNK = 2

_P = (5.621959008883515e-07, 0.9999574870750662, -0.4992065685478449,
      0.32697310001386687, -0.2228362583280196, 0.13076503250423846,
      -0.052624851367851076, 0.010119082927824848)


def _softplus16(x, mv):
    t = jnp.exp(-jnp.abs(x))
    p = jnp.full((16,), _P[7], jnp.float32)
    for k in range(6, -1, -1):
        p = p * t + _P[k]
    return jnp.maximum(x, 0.0) + p + mv


def _sc_body(mv_hbm, x_hbm, o_hbm, mv_v, buf, sem_in, sem_out):
    wid = lax.axis_index("s") * 2 + lax.axis_index("c")
    base = wid * _PER_W

    pltpu.sync_copy(mv_hbm, mv_v)
    mvv = mv_v[...]
    t = jnp.exp(-jnp.abs(mvv))
    p = jnp.full((16,), _P[7], jnp.float32)
    for k in range(6, -1, -1):
        p = p * t + _P[k]
    mv = jnp.maximum(mvv, 0.0) + p

    def chunk_body(ci, _):
        s0 = base + ci * _CHUNK
        pltpu.async_copy(x_hbm.at[pl.ds(s0, _CHUNK)], buf, sem_in).wait()

        for cc in range(_CHUNK):
            def row_body(r, _, cc=cc):
                def vec_body(j, _):
                    x = buf[cc, r, pl.ds(j * 16, 16)]
                    buf[cc, r, pl.ds(j * 16, 16)] = _softplus16(x, mv)
                    return 0
                return lax.fori_loop(0, 12, vec_body, 0, unroll=4)

            lax.fori_loop(0, _ROWS, row_body, 0)
        pltpu.async_copy(buf, o_hbm.at[pl.ds(s0, _CHUNK)], sem_out).wait()
        return 0

    lax.fori_loop(0, _PER_W // _CHUNK, chunk_body, 0)


def kernel(input_, _min_value):
    n, c, h, w = input_.shape
    xt = jnp.transpose(input_, (0, 2, 3, 1)).reshape(_S, _ROWS, _C)
    mv16 = jnp.broadcast_to(jnp.asarray(_min_value, jnp.float32).reshape(1), (16,))

    mesh = plsc.VectorSubcoreMesh(core_axis_name="c", subcore_axis_name="s")
    fn = pl.kernel(
        _sc_body,
        out_type=jax.ShapeDtypeStruct((_S, _ROWS, _C), jnp.float32),
        mesh=mesh,
        scratch_types=[
            pltpu.VMEM((16,), jnp.float32),
            pltpu.VMEM((_CHUNK, _ROWS, _C), jnp.float32),
            pltpu.SemaphoreType.DMA,
            pltpu.SemaphoreType.DMA,
        ],
        compiler_params=pltpu.CompilerParams(use_tc_tiling_on_sc=True),
    )
    out = fn(mv16, xt)
    return jnp.transpose(out.reshape(n, h, w, c), (0, 3, 1, 2))


# speedup vs baseline: 3.4190x; 3.4190x over previous
import jax
import jax.numpy as jnp
from jax import lax
from jax.experimental import pallas as pl
from jax.experimental.pallas import tpu as pltpu
from jax.experimental.pallas import tpu_sc as plsc

_S = 896
_ROWS = 56
_C = 384
_NW = 32
_PER_W = _S // _NW
_CHUNK = 2
_NCHUNK = _PER_W // _CHUNK

_P = (5.621959008883515e-07, 0.9999574870750662, -0.4992065685478449,
      0.32697310001386687, -0.2228362583280196, 0.13076503250423846,
      -0.052624851367851076, 0.010119082927824848)


def _softplus16(x, mv):
    t = jnp.exp(-jnp.abs(x))
    p = jnp.full((16,), _P[7], jnp.float32)
    for k in range(6, -1, -1):
        p = p * t + _P[k]
    return jnp.maximum(x, 0.0) + p + mv


def _transform_chunk(buf, mv):
    for cc in range(_CHUNK):
        def row_body(r, _, cc=cc):
            for j in range(12):
                x = buf[cc, r, pl.ds(j * 16, 16)]
                buf[cc, r, pl.ds(j * 16, 16)] = _softplus16(x, mv)
            return 0
        lax.fori_loop(0, _ROWS, row_body, 0, unroll=2)


def _sc_body(mv_hbm, x_hbm, o_hbm, mv_v, buf0, buf1, si0, si1, so0, so1):
    wid = lax.axis_index("s") * 2 + lax.axis_index("c")
    base = wid * _PER_W

    pltpu.sync_copy(mv_hbm, mv_v)
    mvv = mv_v[...]
    t = jnp.exp(-jnp.abs(mvv))
    p = jnp.full((16,), _P[7], jnp.float32)
    for k in range(6, -1, -1):
        p = p * t + _P[k]
    mv = jnp.maximum(mvv, 0.0) + p

    bufs = (buf0, buf1)
    sin = (si0, si1)
    sout = (so0, so1)

    def in_copy(ci, b):
        return pltpu.make_async_copy(
            x_hbm.at[pl.ds(base + ci * _CHUNK, _CHUNK)], bufs[b], sin[b])

    def out_copy(ci, b):
        return pltpu.make_async_copy(
            bufs[b], o_hbm.at[pl.ds(base + ci * _CHUNK, _CHUNK)], sout[b])

    in_copy(0, 0).start()
    for ci in range(_NCHUNK):
        cur = ci & 1
        nxt = cur ^ 1
        if ci + 1 < _NCHUNK:
            if ci >= 1:
                out_copy(ci - 1, nxt).wait()
            in_copy(ci + 1, nxt).start()
        in_copy(ci, cur).wait()
        _transform_chunk(bufs[cur], mv)
        out_copy(ci, cur).start()
    out_copy(_NCHUNK - 2, 0).wait()
    out_copy(_NCHUNK - 1, 1).wait()


def kernel(input_, _min_value):
    n, c, h, w = input_.shape
    xt = jnp.transpose(input_, (0, 2, 3, 1)).reshape(_S, _ROWS, _C)
    mv16 = jnp.broadcast_to(jnp.asarray(_min_value, jnp.float32).reshape(1), (16,))

    mesh = plsc.VectorSubcoreMesh(core_axis_name="c", subcore_axis_name="s")
    fn = pl.kernel(
        _sc_body,
        out_type=jax.ShapeDtypeStruct((_S, _ROWS, _C), jnp.float32),
        mesh=mesh,
        scratch_types=[
            pltpu.VMEM((16,), jnp.float32),
            pltpu.VMEM((_CHUNK, _ROWS, _C), jnp.float32),
            pltpu.VMEM((_CHUNK, _ROWS, _C), jnp.float32),
            pltpu.SemaphoreType.DMA,
            pltpu.SemaphoreType.DMA,
            pltpu.SemaphoreType.DMA,
            pltpu.SemaphoreType.DMA,
        ],
        compiler_params=pltpu.CompilerParams(use_tc_tiling_on_sc=True),
    )
    out = fn(mv16, xt)
    return jnp.transpose(out.reshape(n, h, w, c), (0, 3, 1, 2))


# speedup vs baseline: 11.7417x; 3.4342x over previous
import jax
import jax.numpy as jnp
from jax.experimental import pallas as pl
from jax.experimental.pallas import tpu as pltpu

_NB = 2
_LOG2E = 1.4426950408889634
_LN2 = 0.6931471805599453


def _body(mv_ref, x_ref, o_ref):
    x = x_ref[...]
    mv = jnp.logaddexp(mv_ref[0], 0.0)
    a = jnp.abs(x)
    m = jnp.maximum(x, 0.0)
    t = jnp.exp2(a * (-_LOG2E))
    sp = m + jnp.log2(1.0 + t) * _LN2 + mv
    ch = jax.lax.broadcasted_iota(jnp.int32, x.shape, 3)
    o_ref[...] = jnp.where(ch < 192, sp, x)


def kernel(input_, _min_value):
    n, c, h, w = input_.shape
    xt = jnp.transpose(input_, (0, 2, 3, 1))
    mv = jnp.asarray(_min_value, jnp.float32).reshape(1)
    out = pl.pallas_call(
        _body,
        grid=(n // _NB,),
        in_specs=[
            pl.BlockSpec(memory_space=pltpu.SMEM),
            pl.BlockSpec((_NB, h, w, c), lambda i: (i, 0, 0, 0)),
        ],
        out_specs=pl.BlockSpec((_NB, h, w, c), lambda i: (i, 0, 0, 0)),
        out_shape=jax.ShapeDtypeStruct((n, h, w, c), input_.dtype),
        compiler_params=pltpu.CompilerParams(
            dimension_semantics=("parallel",),
        ),
    )(mv, xt)
    return jnp.transpose(out, (0, 3, 1, 2))
